# baseline (speedup 1.0000x reference)
import jax
import jax.numpy as jnp
from jax import lax
from jax.experimental import pallas as pl
from jax.experimental.pallas import tpu as pltpu
from jax.experimental.pallas import tpu_sc as plsc

D = 16
NB = 16384
NT = 200
NW = 32
BW = NB // NW
NBT = BW // 128
HT = NBT * 1024
TSTRIDE = 2 * (NB // 128) * 8 * 128
FSTRIDE = (NB // 128) * 8 * 128
TCH = 16
TOFFS = [16 * c for c in range(12)] + [NT - TCH]


def _embed_body(table_hbm, x_hbm, out_hbm,
                xa, xb, idx_t, rows0, rows1, rt0, rt1,
                xs0, xs1, g0, g1, o0, o1):
    wid = lax.axis_index("s") * 2 + lax.axis_index("c")
    bbase = wid * BW
    iot = lax.iota(jnp.int32, 16)
    cvec = (iot // 8) * HT + (iot % 8) * 128
    tvec = iot * BW
    x_b = [xa, xb]
    rows_b = [rows0, rows1]
    rt_b = [rt0, rt1]
    g_b = [g0, g1]
    o_b = [o0, o1]
    xs_b = [xs0, xs1]

    def slab_copy(c, start):
        src = x_hbm.at[pl.ds(bbase, BW), pl.ds(TOFFS[c], TCH)]
        if start:
            pltpu.async_copy(src, x_b[c % 2], xs_b[c % 2])
        else:
            pltpu.make_async_copy(src, x_b[c % 2], xs_b[c % 2]).wait()

    def gather(tt, p, start):
        src = table_hbm.at[idx_t.at[pl.ds(tt * BW, BW)]]
        if start:
            pltpu.async_copy(src, rows_b[p], g_b[p])
        else:
            pltpu.make_async_copy(src, rows_b[p], g_b[p]).wait()

    def out_copy(t, p, start):
        obase = t * TSTRIDE + wid * HT
        src0, src1 = rt_b[p].at[pl.ds(0, HT)], rt_b[p].at[pl.ds(HT, HT)]
        dst0 = out_hbm.at[pl.ds(obase, HT)]
        dst1 = out_hbm.at[pl.ds(obase + FSTRIDE, HT)]
        if start:
            pltpu.async_copy(src0, dst0, o_b[p])
            pltpu.async_copy(src1, dst1, o_b[p])
        else:
            pltpu.make_async_copy(src0, dst0, o_b[p]).wait()
            pltpu.make_async_copy(src1, dst1, o_b[p]).wait()

    slab_copy(0, start=True)

    for c in range(len(TOFFS)):
        xcur = x_b[c % 2]
        toff = TOFFS[c]
        slab_copy(c, start=False)
        if c + 1 < len(TOFFS):
            slab_copy(c + 1, start=True)

        @plsc.parallel_loop(0, BW // 16, unroll=2)
        def _idx_transpose(g, xcur=xcur):
            rbase = g * 16
            vs = [xcur[rbase + j] for j in range(16)]
            for j in range(16):
                plsc.store_scatter(idx_t, [tvec + (rbase + j)], vs[j])

        gather(0, 0, start=True)

        def step(tt, p, q, toff=toff):
            t = toff + tt

            @pl.when(tt < TCH - 1)
            def _prefetch():
                gather(tt + 1, q, start=True)

            gather(tt, p, start=False)

            @pl.when(t >= 2)
            def _drain():
                out_copy(t - 2, p, start=False)

            @plsc.parallel_loop(0, BW // 16, unroll=2)
            def _transpose(g):
                idxbase = cvec + ((g // 8) * 1024 + (g % 8) * 16)
                rbase = g * 16
                vs = [rows_b[p][rbase + j] for j in range(16)]
                for j in range(16):
                    plsc.store_scatter(rt_b[p], [idxbase + j], vs[j])

            out_copy(t, p, start=True)

        def per_i(i, carry, step=step):
            step(2 * i, 0, 1)
            step(2 * i + 1, 1, 0)
            return carry

        lax.fori_loop(0, TCH // 2, per_i, 0)

    out_copy(NT - 2, (NT - 2) % 2, start=False)
    out_copy(NT - 1, (NT - 1) % 2, start=False)


@jax.jit
def kernel(x, table):
    mesh = plsc.VectorSubcoreMesh(core_axis_name="c", subcore_axis_name="s")
    out = pl.kernel(
        _embed_body,
        out_type=jax.ShapeDtypeStruct((NT * TSTRIDE,), jnp.float32),
        mesh=mesh,
        scratch_types=[
            pltpu.VMEM((BW, TCH), jnp.int32),
            pltpu.VMEM((BW, TCH), jnp.int32),
            pltpu.VMEM((TCH * BW,), jnp.int32),
            pltpu.VMEM((BW, D), jnp.float32),
            pltpu.VMEM((BW, D), jnp.float32),
            pltpu.VMEM((2 * HT,), jnp.float32),
            pltpu.VMEM((2 * HT,), jnp.float32),
            pltpu.SemaphoreType.DMA,
            pltpu.SemaphoreType.DMA,
            pltpu.SemaphoreType.DMA,
            pltpu.SemaphoreType.DMA,
            pltpu.SemaphoreType.DMA,
            pltpu.SemaphoreType.DMA,
        ],
        compiler_params=pltpu.CompilerParams(
            use_tc_tiling_on_sc=False, needs_layout_passes=False),
    )(table, x.astype(jnp.int32))
    out = out.reshape(NT, 2, NB // 128, 8, 128)
    return out.transpose(2, 4, 0, 1, 3).reshape(NB, NT, D)


# speedup vs baseline: 1.0896x; 1.0896x over previous
import jax
import jax.numpy as jnp
from jax import lax
from jax.experimental import pallas as pl
from jax.experimental.pallas import tpu as pltpu
from jax.experimental.pallas import tpu_sc as plsc

D = 16
NB = 16384
NT = 200
NW = 32
BW = NB // NW
NBT = BW // 128
HT = NBT * 1024
TSTRIDE = 2 * (NB // 128) * 8 * 128
FSTRIDE = (NB // 128) * 8 * 128


def _embed_body(table_hbm, idx_hbm, out_hbm,
                idx0, idx1, rows0, rows1, rt0, rt1,
                g0, g1, o0, o1, i0, i1):
    wid = lax.axis_index("s") * 2 + lax.axis_index("c")
    bbase = wid * BW
    iot = lax.iota(jnp.int32, 16)
    cvec = (iot // 8) * HT + (iot % 8) * 128
    idx_b = [idx0, idx1]
    rows_b = [rows0, rows1]
    rt_b = [rt0, rt1]
    g_b = [g0, g1]
    o_b = [o0, o1]
    i_b = [i0, i1]

    def idx_copy(t, p, start):
        src = idx_hbm.at[pl.ds(t * NB + bbase, BW)]
        if start:
            pltpu.async_copy(src, idx_b[p], i_b[p])
        else:
            pltpu.make_async_copy(src, idx_b[p], i_b[p]).wait()

    def out_copy(t, p, start):
        obase = t * TSTRIDE + wid * HT
        src0, src1 = rt_b[p].at[pl.ds(0, HT)], rt_b[p].at[pl.ds(HT, HT)]
        dst0 = out_hbm.at[pl.ds(obase, HT)]
        dst1 = out_hbm.at[pl.ds(obase + FSTRIDE, HT)]
        if start:
            pltpu.async_copy(src0, dst0, o_b[p])
            pltpu.async_copy(src1, dst1, o_b[p])
        else:
            pltpu.make_async_copy(src0, dst0, o_b[p]).wait()
            pltpu.make_async_copy(src1, dst1, o_b[p]).wait()

    pltpu.sync_copy(idx_hbm.at[pl.ds(bbase, BW)], idx0)
    pltpu.async_copy(table_hbm.at[idx0], rows0, g0)
    idx_copy(1, 1, start=True)

    def step(t, p, q):
        @pl.when(t < NT - 1)
        def _prefetch():
            idx_copy(t + 1, q, start=False)
            pltpu.async_copy(table_hbm.at[idx_b[q]], rows_b[q], g_b[q])

        pltpu.make_async_copy(table_hbm.at[idx_b[p]], rows_b[p], g_b[p]).wait()

        @pl.when(t < NT - 2)
        def _iprefetch():
            idx_copy(t + 2, p, start=True)

        @pl.when(t >= 2)
        def _drain():
            out_copy(t - 2, p, start=False)

        @plsc.parallel_loop(0, BW // 16, unroll=2)
        def _transpose(g):
            idxbase = cvec + ((g // 8) * 1024 + (g % 8) * 16)
            rbase = g * 16
            vs = [rows_b[p][rbase + j] for j in range(16)]
            for j in range(16):
                plsc.store_scatter(rt_b[p], [idxbase + j], vs[j])

        out_copy(t, p, start=True)

    def per_i(i, carry):
        step(2 * i, 0, 1)
        step(2 * i + 1, 1, 0)
        return carry

    lax.fori_loop(0, NT // 2, per_i, 0)

    out_copy(NT - 2, (NT - 2) % 2, start=False)
    out_copy(NT - 1, (NT - 1) % 2, start=False)


@jax.jit
def kernel(x, table):
    idx = x.T.reshape(-1).astype(jnp.int32)
    mesh = plsc.VectorSubcoreMesh(core_axis_name="c", subcore_axis_name="s")
    out = pl.kernel(
        _embed_body,
        out_type=jax.ShapeDtypeStruct((NT * TSTRIDE,), jnp.float32),
        mesh=mesh,
        scratch_types=[
            pltpu.VMEM((BW,), jnp.int32),
            pltpu.VMEM((BW,), jnp.int32),
            pltpu.VMEM((BW, D), jnp.float32),
            pltpu.VMEM((BW, D), jnp.float32),
            pltpu.VMEM((2 * HT,), jnp.float32),
            pltpu.VMEM((2 * HT,), jnp.float32),
            pltpu.SemaphoreType.DMA,
            pltpu.SemaphoreType.DMA,
            pltpu.SemaphoreType.DMA,
            pltpu.SemaphoreType.DMA,
            pltpu.SemaphoreType.DMA,
            pltpu.SemaphoreType.DMA,
        ],
        compiler_params=pltpu.CompilerParams(
            use_tc_tiling_on_sc=False, needs_layout_passes=False),
    )(table, idx)
    out = out.reshape(NT, 2, NB // 128, 8, 128)
    return out.transpose(2, 4, 0, 1, 3).reshape(NB, NT, D)
